# baseline (speedup 1.0000x reference)
import functools

import jax
import jax.numpy as jnp
from jax import lax
from jax.experimental import pallas as pl
from jax.experimental.pallas import tpu as pltpu
from jax.experimental.pallas import tpu_sc as plsc

N = 10000
E = 320000
F = 128
FH = F // 2

NC = 2
NS = 16
L = 16
EPT = E // NS
CH = 400
NCHUNK = EPT // CH
SCH = 5
NSUP = NCHUNK // SCH
GRP = CH // L
NBUF = 2
NPAD = 10240
DCH = NPAD // NS
NROW = NPAD // NS

_mesh = plsc.VectorSubcoreMesh(
    core_axis_name="c", subcore_axis_name="s", num_cores=NC, num_subcores=NS
)


@functools.partial(
    pl.kernel,
    out_type=(
        jax.ShapeDtypeStruct((NC, NPAD, FH), jnp.float32),
        jax.ShapeDtypeStruct((NPAD,), jnp.float32),
    ),
    mesh=_mesh,
    compiler_params=pltpu.CompilerParams(
        needs_layout_passes=False, use_tc_tiling_on_sc=False),
    scratch_types=(
        pltpu.VMEM((N,), jnp.float32),
        pltpu.VMEM((EPT,), jnp.int32),
        pltpu.VMEM((SCH, CH), jnp.int32),
        pltpu.VMEM((SCH * CH,), jnp.int32),
        pltpu.VMEM((SCH * CH,), jnp.float32),
        pltpu.VMEM((SCH * CH,), jnp.float32),
        pltpu.VMEM((NBUF, CH, FH), jnp.float32),
        pltpu.SemaphoreType.DMA,
        pltpu.SemaphoreType.DMA,
        pltpu.SemaphoreType.DMA,
        pltpu.SemaphoreType.DMA,
        pltpu.VMEM_SHARED((NPAD, FH), jnp.float32),
        pltpu.VMEM_SHARED((NPAD,), jnp.float32),
    ),
)
def _gat_edges_sc(row_hbm, col_hbm, s_hbm, t2_hbm, fsp_hbm,
                  outp_hbm, den_hbm,
                  s_v, col_v, rowb_v, rowf_v, tvb_v, pb_v, rows_v,
                  sem_rows, sem_tv, sem_sc, sem_den, out_sh, den_sh):
    cid = lax.axis_index("c")
    sid = lax.axis_index("s")

    pltpu.sync_copy(col_hbm.at[sid], col_v)
    pltpu.sync_copy(s_hbm, s_v)

    def _zrow(i, c):
        for j in range(FH // L):
            rows_v[0, i, pl.ds(j * L, L)] = jnp.zeros((L,), jnp.float32)
        return c
    lax.fori_loop(0, CH, _zrow, 0)

    def _zden(i, c):
        tvb_v[pl.ds(i * L, L)] = jnp.zeros((L,), jnp.float32)
        return c
    lax.fori_loop(0, DCH // L, _zden, 0)

    pltpu.sync_copy(rows_v.at[0], out_sh.at[pl.ds(sid * NROW, CH)])
    pltpu.sync_copy(rows_v.at[0, pl.ds(0, NROW - CH)],
                    out_sh.at[pl.ds(sid * NROW + CH, NROW - CH)])

    @pl.when(cid == 0)
    def _():
        pltpu.sync_copy(tvb_v.at[pl.ds(0, DCH)],
                        den_sh.at[pl.ds(sid * DCH, DCH)])

    coff = cid * N

    def _adj(i, c):
        col_v[pl.ds(i * L, L)] = col_v[pl.ds(i * L, L)] + coff
        return c
    lax.fori_loop(0, EPT // L, _adj, 0)

    plsc.subcore_barrier()

    def _issue(k, buf):
        pltpu.async_copy(fsp_hbm.at[col_v.at[pl.ds(k * CH, CH)]],
                         rows_v.at[buf], sem_rows)

    _issue(0, 0)

    def _sup(kk, c):
        @pl.when(kk > 0)
        def _():
            kprev = kk * SCH - 1
            pltpu.make_async_copy(
                rows_v.at[lax.rem(kprev, NBUF)],
                out_sh.at[rowb_v.at[SCH - 1]], sem_sc).wait()

            @pl.when(cid == 0)
            def _():
                pltpu.make_async_copy(pb_v, den_sh.at[rowf_v], sem_den).wait()

        pltpu.sync_copy(row_hbm.at[sid, kk], rowb_v)

        def _flat(i, cc):
            for j in range(SCH):
                rowf_v[pl.ds(j * CH + i * L, L)] = rowb_v[j, pl.ds(i * L, L)]
            return cc
        lax.fori_loop(0, CH // L, _flat, 0)
        pltpu.async_copy(
            t2_hbm.at[col_v.at[pl.ds(kk * (SCH * CH), SCH * CH)]],
            tvb_v, sem_tv).wait()

        for j in range(SCH):
            k = kk * SCH + j
            buf = lax.rem(k, NBUF)
            pltpu.make_async_copy(fsp_hbm.at[col_v.at[pl.ds(k * CH, CH)]],
                                  rows_v.at[buf], sem_rows).wait()
            if j > 0:
                pltpu.make_async_copy(
                    rows_v.at[lax.rem(k - 1, NBUF)],
                    out_sh.at[rowb_v.at[j - 1]], sem_sc).wait()

            @pl.when(k + 1 < NCHUNK)
            def _():
                _issue(k + 1, lax.rem(k + 1, NBUF))

            def _pgrp(g, cc):
                ridx = rowb_v[j, pl.ds(g * L, L)]
                e = (plsc.load_gather(s_v, [ridx])
                     + tvb_v[pl.ds(j * CH + g * L, L)])
                e = jnp.where(e > 0, e, 0.2 * e)
                pb_v[pl.ds(j * CH + g * L, L)] = jnp.exp(e)
                return cc
            lax.fori_loop(0, GRP, _pgrp, 0)

            def _sgrp(g, cc):
                pvec = pb_v[pl.ds(j * CH + g * L, L)]
                for i in range(L):
                    eidx = g * L + i
                    ps = jnp.broadcast_to(pvec[i], (L,))
                    for jj in range(FH // L):
                        rows_v[buf, eidx, pl.ds(jj * L, L)] = (
                            rows_v[buf, eidx, pl.ds(jj * L, L)] * ps)
                return cc
            lax.fori_loop(0, GRP, _sgrp, 0)

            pltpu.async_copy(rows_v.at[buf], out_sh.at[rowb_v.at[j]],
                             sem_sc, add=True)

        @pl.when(cid == 0)
        def _():
            pltpu.async_copy(pb_v, den_sh.at[rowf_v], sem_den, add=True)
        return c
    lax.fori_loop(0, NSUP, _sup, 0)

    pltpu.make_async_copy(
        rows_v.at[lax.rem(NCHUNK - 1, NBUF)],
        out_sh.at[rowb_v.at[SCH - 1]], sem_sc).wait()

    @pl.when(cid == 0)
    def _():
        pltpu.make_async_copy(pb_v, den_sh.at[rowf_v], sem_den).wait()

    plsc.subcore_barrier()

    pltpu.sync_copy(out_sh.at[pl.ds(sid * NROW, NROW)],
                    outp_hbm.at[cid, pl.ds(sid * NROW, NROW)])

    @pl.when(cid == 0)
    def _():
        pltpu.sync_copy(den_sh.at[pl.ds(sid * DCH, DCH)],
                        den_hbm.at[pl.ds(sid * DCH, DCH)])


def _proj_body(x_ref, w_ref, a_ref, fsp_ref, st_ref):
    f = jnp.dot(x_ref[:], w_ref[:], preferred_element_type=jnp.float32)
    fsp_ref[pl.ds(0, N), :] = f[:, :FH]
    fsp_ref[pl.ds(N, N), :] = f[:, FH:]
    st_ref[:] = lax.dot_general(a_ref[:], f, (((1,), (1,)), ((), ())),
                                preferred_element_type=jnp.float32)


_proj = pl.pallas_call(
    _proj_body,
    out_shape=[
        jax.ShapeDtypeStruct((NC * N, FH), jnp.float32),
        jax.ShapeDtypeStruct((2, N), jnp.float32),
    ],
)


def _comb_body(p_ref, d_ref, b_ref, o_ref):
    d = d_ref[:, 0] + 1e-16
    o_ref[:] = (jnp.concatenate([p_ref[0], p_ref[1]], axis=1)
                / d[:, None]) + b_ref[:]


_CMB_BLK = 1000
_combine = pl.pallas_call(
    _comb_body,
    grid=(N // _CMB_BLK,),
    in_specs=[
        pl.BlockSpec((2, _CMB_BLK, FH), lambda i: (0, i, 0)),
        pl.BlockSpec((_CMB_BLK, 1), lambda i: (i, 0)),
        pl.BlockSpec((1, F), lambda i: (0, 0)),
    ],
    out_specs=pl.BlockSpec((_CMB_BLK, F), lambda i: (i, 0)),
    out_shape=jax.ShapeDtypeStruct((N, F), jnp.float32),
)


def kernel(x, edge_index, W, b, a_self, a_neigh):
    row = edge_index[0].astype(jnp.int32).reshape(NS, NSUP, SCH, CH)
    col = edge_index[1].astype(jnp.int32).reshape(NS, EPT)
    A = jnp.stack([a_self, a_neigh])
    fsp, st = _proj(x, W, A)
    s = st[0]
    t2 = jnp.concatenate([st[1], st[1]])
    outp, den = _gat_edges_sc(row, col, s, t2, fsp)
    return _combine(outp, den.reshape(NPAD, 1), b.reshape(1, F))


# speedup vs baseline: 1.9814x; 1.9814x over previous
import functools

import jax
import jax.numpy as jnp
from jax import lax
from jax.experimental import pallas as pl
from jax.experimental.pallas import tpu as pltpu
from jax.experimental.pallas import tpu_sc as plsc

N = 10000
E = 320000
F = 128
FH = F // 2

NC = 2
NS = 16
L = 16
EPT = E // NS
CH = 400
NCHUNK = EPT // CH
SCH = 5
NSUP = NCHUNK // SCH
GRP = CH // L
NBUF = 2
NPAD = 10240
DCH = NPAD // NS
NROW = NPAD // NS

_mesh = plsc.VectorSubcoreMesh(
    core_axis_name="c", subcore_axis_name="s", num_cores=NC, num_subcores=NS
)


@functools.partial(
    pl.kernel,
    out_type=(
        jax.ShapeDtypeStruct((NC, NPAD, FH), jnp.float32),
        jax.ShapeDtypeStruct((NPAD,), jnp.float32),
    ),
    mesh=_mesh,
    compiler_params=pltpu.CompilerParams(
        needs_layout_passes=False, use_tc_tiling_on_sc=False),
    scratch_types=(
        pltpu.VMEM((N,), jnp.float32),
        pltpu.VMEM((EPT,), jnp.int32),
        pltpu.VMEM((SCH, CH), jnp.int32),
        pltpu.VMEM((SCH * CH,), jnp.int32),
        pltpu.VMEM((SCH * CH,), jnp.float32),
        pltpu.VMEM((SCH * CH,), jnp.float32),
        pltpu.VMEM((NBUF, CH, FH), jnp.float32),
        pltpu.SemaphoreType.DMA,
        pltpu.SemaphoreType.DMA,
        pltpu.SemaphoreType.DMA,
        pltpu.SemaphoreType.DMA,
        pltpu.VMEM_SHARED((NPAD, FH), jnp.float32),
        pltpu.VMEM_SHARED((NPAD,), jnp.float32),
    ),
)
def _gat_edges_sc(row_hbm, col_hbm, s_hbm, t2_hbm, fsp_hbm,
                  outp_hbm, den_hbm,
                  s_v, col_v, rowb_v, rowf_v, tvb_v, pb_v, rows_v,
                  sem_rows, sem_tv, sem_sc, sem_den, out_sh, den_sh):
    cid = lax.axis_index("c")
    sid = lax.axis_index("s")

    pltpu.sync_copy(col_hbm.at[sid], col_v)
    pltpu.sync_copy(s_hbm, s_v)

    def _zrow(i, c):
        for j in range(FH // L):
            rows_v[0, i, pl.ds(j * L, L)] = jnp.zeros((L,), jnp.float32)
        return c
    lax.fori_loop(0, CH, _zrow, 0)

    def _zden(i, c):
        tvb_v[pl.ds(i * L, L)] = jnp.zeros((L,), jnp.float32)
        return c
    lax.fori_loop(0, DCH // L, _zden, 0)

    pltpu.sync_copy(rows_v.at[0], out_sh.at[pl.ds(sid * NROW, CH)])
    pltpu.sync_copy(rows_v.at[0, pl.ds(0, NROW - CH)],
                    out_sh.at[pl.ds(sid * NROW + CH, NROW - CH)])

    @pl.when(cid == 0)
    def _():
        pltpu.sync_copy(tvb_v.at[pl.ds(0, DCH)],
                        den_sh.at[pl.ds(sid * DCH, DCH)])

    coff = cid * N

    def _adj(i, c):
        col_v[pl.ds(i * L, L)] = col_v[pl.ds(i * L, L)] + coff
        return c
    lax.fori_loop(0, EPT // L, _adj, 0)

    plsc.subcore_barrier()

    def _issue(k, buf):
        pltpu.async_copy(fsp_hbm.at[col_v.at[pl.ds(k * CH, CH)]],
                         rows_v.at[buf], sem_rows)

    _issue(0, 0)

    def _sup(kk, c):
        @pl.when(kk > 0)
        def _():
            kprev = kk * SCH - 1
            pltpu.make_async_copy(
                rows_v.at[lax.rem(kprev, NBUF)],
                out_sh.at[rowb_v.at[SCH - 1]], sem_sc).wait()

            @pl.when(cid == 0)
            def _():
                pltpu.make_async_copy(pb_v, den_sh.at[rowf_v], sem_den).wait()

        pltpu.sync_copy(row_hbm.at[sid, kk], rowb_v)

        def _flat(i, cc):
            for j in range(SCH):
                rowf_v[pl.ds(j * CH + i * L, L)] = rowb_v[j, pl.ds(i * L, L)]
            return cc
        lax.fori_loop(0, CH // L, _flat, 0)
        pltpu.async_copy(
            t2_hbm.at[col_v.at[pl.ds(kk * (SCH * CH), SCH * CH)]],
            tvb_v, sem_tv).wait()

        for j in range(SCH):
            k = kk * SCH + j
            buf = lax.rem(k, NBUF)
            pltpu.make_async_copy(fsp_hbm.at[col_v.at[pl.ds(k * CH, CH)]],
                                  rows_v.at[buf], sem_rows).wait()
            if j > 0:
                pltpu.make_async_copy(
                    rows_v.at[lax.rem(k - 1, NBUF)],
                    out_sh.at[rowb_v.at[j - 1]], sem_sc).wait()

            @pl.when(k + 1 < NCHUNK)
            def _():
                _issue(k + 1, lax.rem(k + 1, NBUF))

            def _pgrp(g, cc):
                ridx = rowb_v[j, pl.ds(g * L, L)]
                e = (plsc.load_gather(s_v, [ridx])
                     + tvb_v[pl.ds(j * CH + g * L, L)])
                e = jnp.where(e > 0, e, 0.2 * e)
                pb_v[pl.ds(j * CH + g * L, L)] = jnp.exp(e)
                return cc
            lax.fori_loop(0, GRP, _pgrp, 0)

            def _sgrp(g, cc):
                pvec = pb_v[pl.ds(j * CH + g * L, L)]
                base = g * L
                nf = FH // L
                vals = [rows_v[buf, base, pl.ds(jj * L, L)]
                        for jj in range(nf)]
                for i in range(L):
                    nxt = ([rows_v[buf, base + i + 1, pl.ds(jj * L, L)]
                            for jj in range(nf)] if i < L - 1 else None)
                    ps = jnp.broadcast_to(pvec[i], (L,))
                    for jj in range(nf):
                        rows_v[buf, base + i, pl.ds(jj * L, L)] = (
                            vals[jj] * ps)
                    vals = nxt
                return cc
            lax.fori_loop(0, GRP, _sgrp, 0)

            pltpu.async_copy(rows_v.at[buf], out_sh.at[rowb_v.at[j]],
                             sem_sc, add=True)

        @pl.when(cid == 0)
        def _():
            pltpu.async_copy(pb_v, den_sh.at[rowf_v], sem_den, add=True)
        return c
    lax.fori_loop(0, NSUP, _sup, 0)

    pltpu.make_async_copy(
        rows_v.at[lax.rem(NCHUNK - 1, NBUF)],
        out_sh.at[rowb_v.at[SCH - 1]], sem_sc).wait()

    @pl.when(cid == 0)
    def _():
        pltpu.make_async_copy(pb_v, den_sh.at[rowf_v], sem_den).wait()

    plsc.subcore_barrier()

    pltpu.sync_copy(out_sh.at[pl.ds(sid * NROW, NROW)],
                    outp_hbm.at[cid, pl.ds(sid * NROW, NROW)])

    @pl.when(cid == 0)
    def _():
        pltpu.sync_copy(den_sh.at[pl.ds(sid * DCH, DCH)],
                        den_hbm.at[pl.ds(sid * DCH, DCH)])


def _proj_body(x_ref, w_ref, a_ref, fsp_ref, st_ref):
    f = jnp.dot(x_ref[:], w_ref[:], preferred_element_type=jnp.float32)
    fsp_ref[pl.ds(0, N), :] = f[:, :FH]
    fsp_ref[pl.ds(N, N), :] = f[:, FH:]
    st_ref[:] = lax.dot_general(a_ref[:], f, (((1,), (1,)), ((), ())),
                                preferred_element_type=jnp.float32)


_proj = pl.pallas_call(
    _proj_body,
    out_shape=[
        jax.ShapeDtypeStruct((NC * N, FH), jnp.float32),
        jax.ShapeDtypeStruct((2, N), jnp.float32),
    ],
)


def _comb_body(p_ref, d_ref, b_ref, o_ref):
    d = d_ref[:, 0] + 1e-16
    o_ref[:] = (jnp.concatenate([p_ref[0], p_ref[1]], axis=1)
                / d[:, None]) + b_ref[:]


_CMB_BLK = 1000
_combine = pl.pallas_call(
    _comb_body,
    grid=(N // _CMB_BLK,),
    in_specs=[
        pl.BlockSpec((2, _CMB_BLK, FH), lambda i: (0, i, 0)),
        pl.BlockSpec((_CMB_BLK, 1), lambda i: (i, 0)),
        pl.BlockSpec((1, F), lambda i: (0, 0)),
    ],
    out_specs=pl.BlockSpec((_CMB_BLK, F), lambda i: (i, 0)),
    out_shape=jax.ShapeDtypeStruct((N, F), jnp.float32),
)


def kernel(x, edge_index, W, b, a_self, a_neigh):
    row = edge_index[0].astype(jnp.int32).reshape(NS, NSUP, SCH, CH)
    col = edge_index[1].astype(jnp.int32).reshape(NS, EPT)
    A = jnp.stack([a_self, a_neigh])
    fsp, st = _proj(x, W, A)
    s = st[0]
    t2 = jnp.concatenate([st[1], st[1]])
    outp, den = _gat_edges_sc(row, col, s, t2, fsp)
    return _combine(outp, den.reshape(NPAD, 1), b.reshape(1, F))
